# Initial kernel scaffold; baseline (speedup 1.0000x reference)
import jax
import jax.numpy as jnp
from jax.experimental import pallas as pl


def kernel(x, codebook, W, b):
    raise NotImplementedError("write your pallas kernel here")


# speedup vs baseline: 18.4798x; 18.4798x over previous
import functools

import jax
import jax.numpy as jnp
from jax.experimental import pallas as pl
from jax.experimental.pallas import tpu as pltpu

NG = 2
NV = 320
GV = NG * NV
VD = 128


def _vq_kernel(x_ref, wt_ref, b_ref, cb_ref, q_ref, cp_ref, pp_ref,
               hist_ref, psum_ref, *, total_rows):
    step = pl.program_id(0)
    nsteps = pl.num_programs(0)

    x = x_ref[...]
    logits = jnp.dot(x, wt_ref[...],
                     preferred_element_type=jnp.float32) + b_ref[...]

    col = jax.lax.broadcasted_iota(jnp.int32, logits.shape, 1)
    g0 = col < NV
    neg = jnp.float32(-jnp.inf)
    m0 = jnp.where(g0, logits, neg)
    m1 = jnp.where(g0, neg, logits)
    mx0 = jnp.max(m0, axis=1, keepdims=True)
    mx1 = jnp.max(m1, axis=1, keepdims=True)
    big = jnp.int32(GV)
    idx0 = jnp.min(jnp.where(m0 == mx0, col, big), axis=1, keepdims=True)
    idx1 = jnp.min(jnp.where(m1 == mx1, col, big), axis=1, keepdims=True)
    oh0 = (col == idx0).astype(jnp.float32)
    oh1 = (col == idx1).astype(jnp.float32)

    e0 = jnp.exp(m0 - mx0)
    e1 = jnp.exp(m1 - mx1)
    p = (e0 / jnp.sum(e0, axis=1, keepdims=True)
         + e1 / jnp.sum(e1, axis=1, keepdims=True))

    hist_inc = jnp.sum(oh0 + oh1, axis=0, keepdims=True)
    psum_inc = jnp.sum(p, axis=0, keepdims=True)

    @pl.when(step == 0)
    def _():
        hist_ref[...] = hist_inc
        psum_ref[...] = psum_inc

    @pl.when(step != 0)
    def _():
        hist_ref[...] += hist_inc
        psum_ref[...] += psum_inc

    q0 = jnp.dot(oh0, cb_ref[...], preferred_element_type=jnp.float32)
    q1 = jnp.dot(oh1, cb_ref[...], preferred_element_type=jnp.float32)
    q_ref[...] = jnp.concatenate([q0, q1], axis=1)

    @pl.when(step == nsteps - 1)
    def _():
        inv = jnp.float32(1.0 / total_rows)
        grow = jax.lax.broadcasted_iota(jnp.int32, (1, GV), 1) < NV

        def pplx(pr):
            t = pr * jnp.log(pr + 1e-7)
            s0 = jnp.sum(jnp.where(grow, t, 0.0))
            s1 = jnp.sum(jnp.where(grow, 0.0, t))
            return jnp.exp(-s0) + jnp.exp(-s1)

        cp_ref[...] = jnp.broadcast_to(pplx(hist_ref[...] * inv), (1, 1))
        pp_ref[...] = jnp.broadcast_to(pplx(psum_ref[...] * inv), (1, 1))


def kernel(x, codebook, W, b):
    bsz, tsz, fsz = x.shape
    xf = x.reshape(-1, fsz)
    rows = xf.shape[0]
    R = 512
    grid = rows // R
    wt = W.T
    cb = codebook[0]
    b2 = b.reshape(1, GV)

    q, cp, pp = pl.pallas_call(
        functools.partial(_vq_kernel, total_rows=rows),
        grid=(grid,),
        in_specs=[
            pl.BlockSpec((R, fsz), lambda i: (i, 0)),
            pl.BlockSpec((fsz, GV), lambda i: (0, 0)),
            pl.BlockSpec((1, GV), lambda i: (0, 0)),
            pl.BlockSpec((GV, VD), lambda i: (0, 0)),
        ],
        out_specs=[
            pl.BlockSpec((R, NG * VD), lambda i: (i, 0)),
            pl.BlockSpec((1, 1), lambda i: (0, 0)),
            pl.BlockSpec((1, 1), lambda i: (0, 0)),
        ],
        out_shape=[
            jax.ShapeDtypeStruct((rows, NG * VD), jnp.float32),
            jax.ShapeDtypeStruct((1, 1), jnp.float32),
            jax.ShapeDtypeStruct((1, 1), jnp.float32),
        ],
        scratch_shapes=[
            pltpu.VMEM((1, GV), jnp.float32),
            pltpu.VMEM((1, GV), jnp.float32),
        ],
    )(xf, wt, b2, cb)

    return (q.reshape(bsz, tsz, NG * VD), codebook.shape[1],
            cp[0, 0], pp[0, 0])
